# baseline (speedup 1.0000x reference)
import functools

import jax
import jax.numpy as jnp
from jax import lax
from jax.experimental import pallas as pl
from jax.experimental.pallas import tpu as pltpu
from jax.experimental.pallas import tpu_sc as plsc

NC, NS = 2, 16
NW = NC * NS
LANES = 16


@functools.lru_cache(maxsize=None)
def _make_sc_gather(b_total: int, hist: int, d_model: int):
    jt = b_total // 128
    j_per_w = jt // NW
    n_units = j_per_w * hist
    n_pairs = n_units // 2
    di = d_model // 8
    assert jt == NW * j_per_w and n_units == 2 * n_pairs
    mesh = plsc.VectorSubcoreMesh(core_axis_name="c", subcore_axis_name="s")

    @functools.partial(
        pl.kernel,
        out_type=jax.ShapeDtypeStruct((hist, di, jt, 8, 128), jnp.float32),
        mesh=mesh,
        scratch_types=[
            pltpu.VMEM((j_per_w * 128 * 128,), jnp.int32),
            pltpu.VMEM((2, 128), jnp.int32),
            pltpu.VMEM((2, 128, d_model), jnp.float32),
            pltpu.VMEM((2, d_model, 128), jnp.float32),
            pltpu.SemaphoreType.DMA,
            pltpu.SemaphoreType.DMA,
            pltpu.SemaphoreType.DMA,
            pltpu.SemaphoreType.DMA,
        ],
        compiler_params=pltpu.CompilerParams(use_tc_tiling_on_sc=False,
                                             needs_layout_passes=False),
    )
    def gather_kernel(x_hbm, table_hbm, out_hbm, idxj, idxcol, g, tp,
                      gs0, gs1, os0, os1):
        wid = lax.axis_index("s") * NC + lax.axis_index("c")
        jbase = wid * j_per_w
        pltpu.sync_copy(x_hbm.at[pl.ds(jbase * 128 * 128, j_per_w * 128 * 128)],
                        idxj)
        iota = lax.iota(jnp.int32, LANES)
        i128 = iota * 128

        def build_idxcol(u, s):
            base = (u // hist) * (128 * 128) + (u % hist)
            for cc in range(8):
                v = plsc.load_gather(idxj, [i128 + (base + cc * 2048)])
                idxcol[s, pl.ds(cc * LANES, LANES)] = v

        def fire_gather(s, gsem):
            pltpu.async_copy(table_hbm.at[idxcol.at[s]], g.at[s], gsem)

        def wait_gather(s, gsem):
            pltpu.make_async_copy(
                table_hbm.at[idxcol.at[s]], g.at[s], gsem).wait()

        def transpose(s):
            def body(d, carry):
                for cc in range(8):
                    v = plsc.load_gather(
                        g.at[s], [iota + cc * LANES, jnp.full((LANES,), d,
                                                              jnp.int32)])
                    tp[s, d, pl.ds(cc * LANES, LANES)] = v
                return carry
            lax.fori_loop(0, d_model, body, 0)

        def fire_out(u, s, osem):
            h = u % hist
            j = jbase + u // hist
            for i in range(di):
                pltpu.async_copy(tp.at[s, pl.ds(i * 8, 8)],
                                 out_hbm.at[h, i, j], osem)

        def wait_out(s, osem):
            pltpu.make_async_copy(
                out_hbm.at[0, 0, 0], tp.at[s, pl.ds(0, 8)], osem).wait()

        build_idxcol(0, 0)
        fire_gather(0, gs0)

        bufs = ((0, gs0, os0), (1, gs1, os1))

        def pair(k, carry):
            u0 = 2 * k
            for s, gsem, osem in bufs:
                u = u0 + s
                other = 1 - s
                ogsem = bufs[other][1]

                @pl.when(u + 1 < n_units)
                def _():
                    build_idxcol(u + 1, other)
                    fire_gather(other, ogsem)

                wait_gather(s, gsem)

                @pl.when(u >= 2)
                def _():
                    for _i in range(di):
                        wait_out(s, osem)

                transpose(s)
                fire_out(u, s, osem)
            return carry

        lax.fori_loop(0, n_pairs, pair, 0)
        for s, _g, osem in bufs:
            for _i in range(di):
                wait_out(s, osem)

    return gather_kernel


def kernel(x, table):
    b, h = x.shape
    d = table.shape[1]
    xp = jnp.pad(x.astype(jnp.int32), ((0, 0), (0, 128 - h))).reshape(-1)
    out5 = _make_sc_gather(b, h, d)(xp, table)
    return out5.transpose(2, 4, 0, 1, 3).reshape(b, h, d)


# speedup vs baseline: 1.1412x; 1.1412x over previous
import functools

import jax
import jax.numpy as jnp
from jax import lax
from jax.experimental import pallas as pl
from jax.experimental.pallas import tpu as pltpu
from jax.experimental.pallas import tpu_sc as plsc

NC, NS = 2, 16
NW = NC * NS
LANES = 16


@functools.lru_cache(maxsize=None)
def _make_sc_gather(b_total: int, hist: int, d_model: int):
    jt = b_total // 128
    j_per_w = jt // NW
    n_units = j_per_w * hist
    n_pairs = n_units // 2
    di = d_model // 8
    assert jt == NW * j_per_w and n_units == 2 * n_pairs
    mesh = plsc.VectorSubcoreMesh(core_axis_name="c", subcore_axis_name="s")

    @functools.partial(
        pl.kernel,
        out_type=jax.ShapeDtypeStruct((hist, di, jt, 8, 128), jnp.float32),
        mesh=mesh,
        scratch_types=[
            pltpu.VMEM((j_per_w * 128 * 128,), jnp.int32),
            pltpu.VMEM((2, 128), jnp.int32),
            pltpu.VMEM((2, 128, d_model), jnp.float32),
            pltpu.VMEM((2, d_model, 128), jnp.float32),
            pltpu.SemaphoreType.DMA,
            pltpu.SemaphoreType.DMA,
            pltpu.SemaphoreType.DMA,
            pltpu.SemaphoreType.DMA,
        ],
        compiler_params=pltpu.CompilerParams(use_tc_tiling_on_sc=False,
                                             needs_layout_passes=False),
    )
    def gather_kernel(x_hbm, table_hbm, out_hbm, idxj, idxcol, g, tp,
                      gs0, gs1, os0, os1):
        wid = lax.axis_index("s") * NC + lax.axis_index("c")
        jbase = wid * j_per_w
        pltpu.sync_copy(x_hbm.at[pl.ds(jbase * 128 * 128, j_per_w * 128 * 128)],
                        idxj)
        iota = lax.iota(jnp.int32, LANES)
        i128 = iota * 128

        def build_idxcol(u, s):
            base = (u // hist) * (128 * 128) + (u % hist)
            for cc in range(8):
                v = plsc.load_gather(idxj, [i128 + (base + cc * 2048)])
                idxcol[s, pl.ds(cc * LANES, LANES)] = v

        def fire_gather(s, gsem):
            pltpu.async_copy(table_hbm.at[idxcol.at[s]], g.at[s], gsem)

        def wait_gather(s, gsem):
            pltpu.make_async_copy(
                table_hbm.at[idxcol.at[s]], g.at[s], gsem).wait()

        svecs = [iota + dd * LANES for dd in range(d_model // LANES)]

        def transpose(s):
            def body(c0, carry):
                for dc in range(8):
                    c = c0 + dc
                    cvec = jnp.full((LANES,), c, jnp.int32)
                    for dd in range(d_model // LANES):
                        v = g[s, c, pl.ds(dd * LANES, LANES)]
                        plsc.store_scatter(tp.at[s], [svecs[dd], cvec], v)
                return carry
            lax.fori_loop(0, 128 // 8, lambda k, c: body(k * 8, c), 0)

        def fire_out(u, s, osem):
            h = u % hist
            j = jbase + u // hist
            for i in range(di):
                pltpu.async_copy(tp.at[s, pl.ds(i * 8, 8)],
                                 out_hbm.at[h, i, j], osem)

        def wait_out(s, osem):
            pltpu.make_async_copy(
                out_hbm.at[0, 0, 0], tp.at[s, pl.ds(0, 8)], osem).wait()

        build_idxcol(0, 0)
        fire_gather(0, gs0)

        bufs = ((0, gs0, os0), (1, gs1, os1))

        def pair(k, carry):
            u0 = 2 * k
            for s, gsem, osem in bufs:
                u = u0 + s
                other = 1 - s
                ogsem = bufs[other][1]

                @pl.when(u + 1 < n_units)
                def _():
                    build_idxcol(u + 1, other)
                    fire_gather(other, ogsem)

                wait_gather(s, gsem)

                @pl.when(u >= 2)
                def _():
                    for _i in range(di):
                        wait_out(s, osem)

                transpose(s)
                fire_out(u, s, osem)
            return carry

        lax.fori_loop(0, n_pairs, pair, 0)
        for s, _g, osem in bufs:
            for _i in range(di):
                wait_out(s, osem)

    return gather_kernel


def kernel(x, table):
    b, h = x.shape
    d = table.shape[1]
    xp = jnp.pad(x.astype(jnp.int32), ((0, 0), (0, 128 - h))).reshape(-1)
    out5 = _make_sc_gather(b, h, d)(xp, table)
    return out5.transpose(2, 4, 0, 1, 3).reshape(b, h, d)
